# baseline (speedup 1.0000x reference)
import functools

import jax
import jax.numpy as jnp
from jax.experimental import pallas as pl
from jax.experimental.pallas import tpu as pltpu


def _phased_lstm_kernel(x_ref, wih_ref, whh_ref, b_ref, wfc_ref, bfc_ref,
                        out_ref, hall_sc, gx_sc, h_sc, c_sc):
    lyr = pl.program_id(1)
    blk = pl.program_id(2)
    n_lyr = pl.num_programs(1)
    n_blk = pl.num_programs(2)

    Tc, Bc, H = x_ref.shape
    H4 = 4 * H

    @pl.when(blk == 0)
    def _():
        h_sc[...] = jnp.zeros_like(h_sc)
        c_sc[...] = jnp.zeros_like(c_sc)

    @pl.when(lyr == 0)
    def _():
        xin = x_ref[...].reshape(Tc * Bc, H)
        g = jnp.dot(xin, wih_ref[0], preferred_element_type=jnp.float32)
        gx_sc[...] = (g + b_ref[0]).reshape(Tc, Bc, H4)

    @pl.when(lyr != 0)
    def _():
        hin = hall_sc[pl.ds(blk * Tc, Tc)].reshape(Tc * Bc, H)
        g = jnp.dot(hin, wih_ref[0], preferred_element_type=jnp.float32)
        gx_sc[...] = (g + b_ref[0]).reshape(Tc, Bc, H4)

    whh = whh_ref[0]

    def one_step(ts, carry):
        h, c = carry
        gates = gx_sc[ts] + jnp.dot(h.astype(jnp.bfloat16), whh,
                                    preferred_element_type=jnp.float32)
        i_g = jax.nn.sigmoid(gates[:, 0 * H:1 * H])
        f_g = jax.nn.sigmoid(gates[:, 1 * H:2 * H])
        g_g = jnp.tanh(gates[:, 2 * H:3 * H])
        o_g = jax.nn.sigmoid(gates[:, 3 * H:4 * H])
        c_new = f_g * c + i_g * g_g
        h_new = o_g * jnp.tanh(c_new)
        hall_sc[blk * Tc + ts] = h_new.astype(jnp.bfloat16)
        return h_new, c_new

    h, c = jax.lax.fori_loop(0, Tc, one_step,
                             (h_sc[...], c_sc[...]), unroll=True)
    h_sc[...] = h
    c_sc[...] = c

    @pl.when(jnp.logical_and(lyr == n_lyr - 1, blk == n_blk - 1))
    def _():
        logits = jnp.dot(h, wfc_ref[...],
                         preferred_element_type=jnp.float32) + bfc_ref[...]
        out_ref[...] = jax.nn.sigmoid(logits).astype(out_ref.dtype)


def _lstm_net(x_tbh, wih, whh, bias, wfc_t, bfc, *, time_block):
    T, B, H = x_tbh.shape
    L = wih.shape[0]
    out_dim = wfc_t.shape[1]

    ncores = 2 if B % 16 == 0 else 1
    Bc = B // ncores

    tb = min(time_block, T)
    while T % tb != 0:
        tb -= 1

    return pl.pallas_call(
        _phased_lstm_kernel,
        out_shape=jax.ShapeDtypeStruct((B, out_dim), jnp.float32),
        grid=(ncores, L, T // tb),
        in_specs=[
            pl.BlockSpec((tb, Bc, H), lambda b, l, t: ((1 - l) * t, b, 0)),
            pl.BlockSpec((1, H, 4 * H), lambda b, l, t: (l, 0, 0)),
            pl.BlockSpec((1, H, 4 * H), lambda b, l, t: (l, 0, 0)),
            pl.BlockSpec((1, 1, 4 * H), lambda b, l, t: (l, 0, 0)),
            pl.BlockSpec((H, out_dim), lambda b, l, t: (0, 0)),
            pl.BlockSpec((1, out_dim), lambda b, l, t: (0, 0)),
        ],
        out_specs=pl.BlockSpec((Bc, out_dim), lambda b, l, t: (b, 0)),
        scratch_shapes=[
            pltpu.VMEM((T, Bc, H), jnp.bfloat16),
            pltpu.VMEM((tb, Bc, 4 * H), jnp.float32),
            pltpu.VMEM((Bc, H), jnp.float32),
            pltpu.VMEM((Bc, H), jnp.float32),
        ],
        compiler_params=pltpu.CompilerParams(
            dimension_semantics=("parallel", "arbitrary", "arbitrary"),
            vmem_limit_bytes=100 * 1024 * 1024),
    )(x_tbh, wih, whh, bias, wfc_t, bfc)


@functools.partial(jax.jit, static_argnames=("time_block",))
def _forward(embedding, lstm0_w_ih, lstm0_w_hh, lstm0_b_ih, lstm0_b_hh,
             lstm1_w_ih, lstm1_w_hh, lstm1_b_ih, lstm1_b_hh,
             fc_w, fc_b, tokens, time_block=16):
    H = embedding.shape[1]
    H4 = 4 * H

    x = jnp.take(embedding.astype(jnp.bfloat16), tokens.T, axis=0)

    wih = jnp.stack([jnp.transpose(lstm0_w_ih),
                     jnp.transpose(lstm1_w_ih)]).astype(jnp.bfloat16)
    whh = jnp.stack([jnp.transpose(lstm0_w_hh),
                     jnp.transpose(lstm1_w_hh)]).astype(jnp.bfloat16)
    bias = jnp.stack([(lstm0_b_ih + lstm0_b_hh).reshape(1, H4),
                      (lstm1_b_ih + lstm1_b_hh).reshape(1, H4)])

    wfc_t = jnp.transpose(fc_w)
    bfc = fc_b.reshape(1, -1)

    out = _lstm_net(x, wih, whh, bias, wfc_t, bfc, time_block=time_block)
    return out.reshape(-1, 1)


def kernel(embedding, lstm0_w_ih, lstm0_w_hh, lstm0_b_ih, lstm0_b_hh,
           lstm1_w_ih, lstm1_w_hh, lstm1_b_ih, lstm1_b_hh,
           fc_w, fc_b, tokens):
    return _forward(embedding, lstm0_w_ih, lstm0_w_hh, lstm0_b_ih, lstm0_b_hh,
                    lstm1_w_ih, lstm1_w_hh, lstm1_b_ih, lstm1_b_hh,
                    fc_w, fc_b, tokens)


# speedup vs baseline: 1.4404x; 1.4404x over previous
import functools

import jax
import jax.numpy as jnp
from jax.experimental import pallas as pl
from jax.experimental.pallas import tpu as pltpu


def _phased_lstm_kernel(x_ref, wih_ref, whh_ref, b_ref, wfc_ref, bfc_ref,
                        out_ref, hall_sc, gx_sc, h_sc, c_sc):
    lyr = pl.program_id(1)
    blk = pl.program_id(2)
    n_lyr = pl.num_programs(1)
    n_blk = pl.num_programs(2)

    Tc, Bc, H = x_ref.shape
    H4 = 4 * H

    @pl.when(blk == 0)
    def _():
        h_sc[...] = jnp.zeros_like(h_sc)
        c_sc[...] = jnp.zeros_like(c_sc)

    @pl.when(lyr == 0)
    def _():
        xin = x_ref[...].reshape(Tc * Bc, H)
        g = jnp.dot(xin, wih_ref[0], preferred_element_type=jnp.float32)
        gx_sc[...] = (g + b_ref[0]).reshape(Tc, Bc, H4)

    @pl.when(lyr != 0)
    def _():
        hin = hall_sc[pl.ds(blk * Tc, Tc)].reshape(Tc * Bc, H)
        g = jnp.dot(hin, wih_ref[0], preferred_element_type=jnp.float32)
        gx_sc[...] = (g + b_ref[0]).reshape(Tc, Bc, H4)

    whh = whh_ref[0]

    def one_step(ts, carry):
        h, c = carry
        gates = gx_sc[ts] + jnp.dot(h.astype(jnp.bfloat16), whh,
                                    preferred_element_type=jnp.float32)
        i_g = jax.nn.sigmoid(gates[:, 0 * H:1 * H])
        f_g = jax.nn.sigmoid(gates[:, 1 * H:2 * H])
        g_g = jnp.tanh(gates[:, 2 * H:3 * H])
        o_g = jax.nn.sigmoid(gates[:, 3 * H:4 * H])
        c_new = f_g * c + i_g * g_g
        h_new = o_g * jnp.tanh(c_new)
        hall_sc[blk * Tc + ts] = h_new.astype(jnp.bfloat16)
        return h_new, c_new

    h, c = jax.lax.fori_loop(0, Tc, one_step,
                             (h_sc[...], c_sc[...]), unroll=True)
    h_sc[...] = h
    c_sc[...] = c

    @pl.when(jnp.logical_and(lyr == n_lyr - 1, blk == n_blk - 1))
    def _():
        logits = jnp.dot(h, wfc_ref[...],
                         preferred_element_type=jnp.float32) + bfc_ref[...]
        out_ref[...] = jax.nn.sigmoid(logits).astype(out_ref.dtype)


def _lstm_net(x_tbh, wih, whh, bias, wfc_t, bfc, *, time_block):
    T, B, H = x_tbh.shape
    L = wih.shape[0]
    out_dim = wfc_t.shape[1]

    ncores = 1
    Bc = B // ncores

    tb = min(time_block, T)
    while T % tb != 0:
        tb -= 1

    return pl.pallas_call(
        _phased_lstm_kernel,
        out_shape=jax.ShapeDtypeStruct((B, out_dim), jnp.float32),
        grid=(ncores, L, T // tb),
        in_specs=[
            pl.BlockSpec((tb, Bc, H), lambda b, l, t: ((1 - l) * t, b, 0)),
            pl.BlockSpec((1, H, 4 * H), lambda b, l, t: (l, 0, 0)),
            pl.BlockSpec((1, H, 4 * H), lambda b, l, t: (l, 0, 0)),
            pl.BlockSpec((1, 1, 4 * H), lambda b, l, t: (l, 0, 0)),
            pl.BlockSpec((H, out_dim), lambda b, l, t: (0, 0)),
            pl.BlockSpec((1, out_dim), lambda b, l, t: (0, 0)),
        ],
        out_specs=pl.BlockSpec((Bc, out_dim), lambda b, l, t: (b, 0)),
        scratch_shapes=[
            pltpu.VMEM((T, Bc, H), jnp.bfloat16),
            pltpu.VMEM((tb, Bc, 4 * H), jnp.float32),
            pltpu.VMEM((Bc, H), jnp.float32),
            pltpu.VMEM((Bc, H), jnp.float32),
        ],
        compiler_params=pltpu.CompilerParams(
            dimension_semantics=("parallel", "arbitrary", "arbitrary"),
            vmem_limit_bytes=100 * 1024 * 1024),
    )(x_tbh, wih, whh, bias, wfc_t, bfc)


@functools.partial(jax.jit, static_argnames=("time_block",))
def _forward(embedding, lstm0_w_ih, lstm0_w_hh, lstm0_b_ih, lstm0_b_hh,
             lstm1_w_ih, lstm1_w_hh, lstm1_b_ih, lstm1_b_hh,
             fc_w, fc_b, tokens, time_block=16):
    H = embedding.shape[1]
    H4 = 4 * H

    x = jnp.take(embedding.astype(jnp.bfloat16), tokens.T, axis=0)

    wih = jnp.stack([jnp.transpose(lstm0_w_ih),
                     jnp.transpose(lstm1_w_ih)]).astype(jnp.bfloat16)
    whh = jnp.stack([jnp.transpose(lstm0_w_hh),
                     jnp.transpose(lstm1_w_hh)]).astype(jnp.bfloat16)
    bias = jnp.stack([(lstm0_b_ih + lstm0_b_hh).reshape(1, H4),
                      (lstm1_b_ih + lstm1_b_hh).reshape(1, H4)])

    wfc_t = jnp.transpose(fc_w)
    bfc = fc_b.reshape(1, -1)

    out = _lstm_net(x, wih, whh, bias, wfc_t, bfc, time_block=time_block)
    return out.reshape(-1, 1)


def kernel(embedding, lstm0_w_ih, lstm0_w_hh, lstm0_b_ih, lstm0_b_hh,
           lstm1_w_ih, lstm1_w_hh, lstm1_b_ih, lstm1_b_hh,
           fc_w, fc_b, tokens):
    return _forward(embedding, lstm0_w_ih, lstm0_w_hh, lstm0_b_ih, lstm0_b_hh,
                    lstm1_w_ih, lstm1_w_hh, lstm1_b_ih, lstm1_b_hh,
                    fc_w, fc_b, tokens)
